# baseline (speedup 1.0000x reference)
import functools

import jax
import jax.numpy as jnp
from jax import lax
from jax.experimental import pallas as pl
from jax.experimental.pallas import tpu as pltpu
from jax.experimental.pallas import tpu_sc as plsc

BATCH = 16384
SEQ = 35
N_BEAKERS = SEQ // 5
IDS_PER_ROW = 4 * N_BEAKERS
D = 64
VOCAB = 7

PAIRS_PER_ROW = IDS_PER_ROW // 2
DP = 2 * D
PVOCAB = VOCAB * VOCAB

NUM_CORES = 2
NUM_SUBCORES = 16
NW = NUM_CORES * NUM_SUBCORES
LANES = 16

B_PER_W = BATCH // NW
PAIRS_PER_W = B_PER_W * PAIRS_PER_ROW

IDX_MINOR = 8 * PAIRS_PER_ROW
IDX_ROWS_PER_W = PAIRS_PER_W // IDX_MINOR
CHUNK_IDX_ROWS = 2
CHUNK_ROWS = CHUNK_IDX_ROWS * IDX_MINOR
CHUNK_B = CHUNK_ROWS // PAIRS_PER_ROW
N_CHUNKS = IDX_ROWS_PER_W // CHUNK_IDX_ROWS
X_SLAB_B = 128
N_SLABS = B_PER_W // X_SLAB_B
IDX_ROWS_PER_SLAB = X_SLAB_B // 8

PT_ROWS_PER_TILE = -(-PVOCAB // NUM_SUBCORES)


def _make_sc_gather():
    mesh = plsc.VectorSubcoreMesh(core_axis_name="c", subcore_axis_name="s")

    @functools.partial(
        pl.kernel,
        mesh=mesh,
        compiler_params=pltpu.CompilerParams(needs_layout_passes=False),
        out_type=jax.ShapeDtypeStruct((BATCH, IDS_PER_ROW * D), jnp.float32),
        scratch_types=[
            pltpu.VMEM_SHARED((PVOCAB, DP), jnp.float32),
            pltpu.VMEM((VOCAB, D), jnp.float32),
            pltpu.VMEM((X_SLAB_B, SEQ), jnp.int32),
            pltpu.VMEM((PAIRS_PER_W,), jnp.int32),
            pltpu.VMEM((3, CHUNK_ROWS, DP), jnp.float32),
            pltpu.SemaphoreType.DMA,
            pltpu.SemaphoreType.DMA,
        ],
    )
    def sc_gather(table_hbm, x_hbm, out_hbm, table_s, t_v, x_v, idx_v, rows_v,
                  sem_g, sem_o):
        sid = lax.axis_index("s")
        wid = sid * NUM_CORES + lax.axis_index("c")
        b_base = wid * B_PER_W

        pltpu.sync_copy(table_hbm, t_v)
        for rr in range(PT_ROWS_PER_TILE):
            r = sid * PT_ROWS_PER_TILE + rr

            @pl.when(r < PVOCAB)
            def _():
                i = (r * 147) >> 10
                j = r - VOCAB * i
                pltpu.sync_copy(t_v.at[i], table_s.at[r, pl.ds(0, D)])
                pltpu.sync_copy(t_v.at[j], table_s.at[r, pl.ds(D, D)])

        def slab(si, carry):
            pltpu.sync_copy(
                x_hbm.at[pl.ds(b_base + si * X_SLAB_B, X_SLAB_B)], x_v)

            def id_step(r, carry2):
                lane = lax.iota(jnp.int32, LANES)
                for k in range(7):
                    g = k * LANES + lane
                    q = (g * 2341) >> 15
                    p = g - PAIRS_PER_ROW * q
                    c0 = 5 * (p >> 1) + 1 + 2 * (p & 1)
                    a = plsc.load_gather(x_v, [r * 8 + q, c0])
                    bb = plsc.load_gather(x_v, [r * 8 + q, c0 + 1])
                    idx_v[pl.ds((si * IDX_ROWS_PER_SLAB + r) * IDX_MINOR
                                + k * LANES, LANES)] = a * VOCAB + bb
                return carry2

            lax.fori_loop(0, IDX_ROWS_PER_SLAB, id_step, 0)
            return carry

        lax.fori_loop(0, N_SLABS, slab, 0)
        plsc.subcore_barrier()

        def gather_descs(ci, buf):
            return [
                pltpu.make_async_copy(
                    table_s.at[idx_v.at[pl.ds(
                        (ci * CHUNK_IDX_ROWS + j) * IDX_MINOR, IDX_MINOR)]],
                    buf.at[pl.ds(j * IDX_MINOR, IDX_MINOR)],
                    sem_g,
                )
                for j in range(CHUNK_IDX_ROWS)
            ]

        def out_desc(ci, buf):
            return pltpu.make_async_copy(
                buf.reshape(CHUNK_B, IDS_PER_ROW * D),
                out_hbm.at[pl.ds(b_base + ci * CHUNK_B, CHUNK_B)], sem_o)

        for d in gather_descs(0, rows_v.at[0]):
            d.start()
        for d in gather_descs(1, rows_v.at[1]):
            d.start()

        def chunk_body(ci, carry):
            buf = rows_v.at[ci % 3]
            nbuf = rows_v.at[(ci + 2) % 3]

            @pl.when(ci >= 1)
            def _():
                out_desc(ci - 1, nbuf).wait()

            @pl.when(ci + 2 < N_CHUNKS)
            def _():
                for d in gather_descs(ci + 2, nbuf):
                    d.start()

            for d in gather_descs(ci, buf):
                d.wait()
            out_desc(ci, buf).start()
            return carry

        lax.fori_loop(0, N_CHUNKS, chunk_body, 0)
        out_desc(N_CHUNKS - 1, rows_v.at[(N_CHUNKS - 1) % 3]).wait()

    return sc_gather


_sc_gather = _make_sc_gather()


def kernel(X, color_table, pos_table):
    del pos_table
    return _sc_gather(color_table, X.astype(jnp.int32))


# speedup vs baseline: 1.0301x; 1.0301x over previous
import functools

import jax
import jax.numpy as jnp
from jax import lax
from jax.experimental import pallas as pl
from jax.experimental.pallas import tpu as pltpu
from jax.experimental.pallas import tpu_sc as plsc

BATCH = 16384
SEQ = 35
N_BEAKERS = SEQ // 5
IDS_PER_ROW = 4 * N_BEAKERS
D = 64
VOCAB = 7

PAIRS_PER_ROW = IDS_PER_ROW // 2
DP = 2 * D
PVOCAB = VOCAB * VOCAB

NUM_CORES = 2
NUM_SUBCORES = 16
NW = NUM_CORES * NUM_SUBCORES
LANES = 16

B_PER_W = BATCH // NW
PAIRS_PER_W = B_PER_W * PAIRS_PER_ROW

IDX_MINOR = 8 * PAIRS_PER_ROW
IDX_ROWS_PER_W = PAIRS_PER_W // IDX_MINOR
CHUNK_IDX_ROWS = 2
CHUNK_ROWS = CHUNK_IDX_ROWS * IDX_MINOR
CHUNK_B = CHUNK_ROWS // PAIRS_PER_ROW
N_CHUNKS = IDX_ROWS_PER_W // CHUNK_IDX_ROWS


def _make_sc_gather():
    mesh = plsc.VectorSubcoreMesh(core_axis_name="c", subcore_axis_name="s")

    @functools.partial(
        pl.kernel,
        mesh=mesh,
        compiler_params=pltpu.CompilerParams(needs_layout_passes=False),
        out_type=jax.ShapeDtypeStruct((BATCH, IDS_PER_ROW * D), jnp.float32),
        scratch_types=[
            pltpu.VMEM_SHARED((PVOCAB, DP), jnp.float32),
            pltpu.VMEM((B_PER_W, SEQ), jnp.int32),
            pltpu.VMEM((PAIRS_PER_W,), jnp.int32),
            pltpu.VMEM((2, CHUNK_ROWS, DP), jnp.float32),
            pltpu.SemaphoreType.DMA,
            pltpu.SemaphoreType.DMA,
        ],
    )
    def sc_gather(table_hbm, x_hbm, out_hbm, table_s, x_v, idx_v, rows_v,
                  sem_g, sem_o):
        wid = lax.axis_index("s") * NUM_CORES + lax.axis_index("c")
        b_base = wid * B_PER_W

        @pl.when(lax.axis_index("s") == 0)
        def _():
            pltpu.sync_copy(table_hbm, table_s)

        pltpu.sync_copy(x_hbm.at[pl.ds(b_base, B_PER_W)], x_v)

        def id_row(r):
            lane = lax.iota(jnp.int32, LANES)
            for k in range(7):
                g = k * LANES + lane
                q = (g * 2341) >> 15
                p = g - PAIRS_PER_ROW * q
                c0 = 5 * (p >> 1) + 1 + 2 * (p & 1)
                b = r * 8 + q
                a = plsc.load_gather(x_v, [b, c0])
                bb = plsc.load_gather(x_v, [b, c0 + 1])
                idx_v[pl.ds(r * IDX_MINOR + k * LANES, LANES)] = a * VOCAB + bb

        def ids_for_chunk(ci):
            for j in range(CHUNK_IDX_ROWS):
                id_row(ci * CHUNK_IDX_ROWS + j)

        def gather_descs(ci, buf):
            return [
                pltpu.make_async_copy(
                    table_s.at[idx_v.at[pl.ds(
                        (ci * CHUNK_IDX_ROWS + j) * IDX_MINOR, IDX_MINOR)]],
                    buf.at[pl.ds(j * IDX_MINOR, IDX_MINOR)],
                    sem_g,
                )
                for j in range(CHUNK_IDX_ROWS)
            ]

        def out_desc(ci, buf):
            return pltpu.make_async_copy(
                buf.reshape(CHUNK_B, IDS_PER_ROW * D),
                out_hbm.at[pl.ds(b_base + ci * CHUNK_B, CHUNK_B)], sem_o)

        ids_for_chunk(0)
        ids_for_chunk(1)
        plsc.subcore_barrier()
        for d in gather_descs(0, rows_v.at[0]):
            d.start()

        def chunk_body(ci, carry):
            buf = rows_v.at[ci % 2]
            nbuf = rows_v.at[(ci + 1) % 2]

            @pl.when(ci + 2 < N_CHUNKS)
            def _():
                ids_for_chunk(ci + 2)

            @pl.when(ci >= 1)
            def _():
                out_desc(ci - 1, nbuf).wait()

            @pl.when(ci + 1 < N_CHUNKS)
            def _():
                for d in gather_descs(ci + 1, nbuf):
                    d.start()

            for d in gather_descs(ci, buf):
                d.wait()
            out_desc(ci, buf).start()
            return carry

        lax.fori_loop(0, N_CHUNKS, chunk_body, 0)
        out_desc(N_CHUNKS - 1, rows_v.at[(N_CHUNKS - 1) % 2]).wait()

    return sc_gather


_sc_gather = _make_sc_gather()


def kernel(X, color_table, pos_table):
    del pos_table
    ptable = jnp.concatenate(
        [jnp.repeat(color_table, VOCAB, axis=0),
         jnp.tile(color_table, (VOCAB, 1))], axis=1)
    return _sc_gather(ptable, X.astype(jnp.int32))
